# Initial kernel scaffold; baseline (speedup 1.0000x reference)
import jax
import jax.numpy as jnp
from jax.experimental import pallas as pl
from jax.experimental.pallas import tpu as pltpu
from jax.experimental.pallas import tpu_sc as plsc

B = 16384
L = 50
D = 64
N = B * L

WINDOW = 128


def kernel(indices, table):
    idx_flat = indices.reshape(1, N).astype(jnp.int32)

    mesh = plsc.VectorSubcoreMesh(core_axis_name="core", subcore_axis_name="subcore")

    @pl.kernel(
        out_type=jax.ShapeDtypeStruct((N, D), table.dtype),
        mesh=mesh,
    )
    def gather_kernel(table_hbm, idx_hbm, out_hbm):
        def body(i_vmem, o_vmem):
            pltpu.sync_copy(table_hbm.at[i_vmem.at[0]], o_vmem)

        pltpu.emit_pipeline(
            body,
            grid=(N // WINDOW,),
            in_specs=[pl.BlockSpec((1, WINDOW), index_map=lambda i: (0, i))],
            out_specs=[pl.BlockSpec((WINDOW, D), index_map=lambda i: (i, 0))],
            core_axis_name=("core", "subcore"),
            dimension_semantics=(pltpu.PARALLEL,),
        )(idx_hbm, out_hbm)

    out = gather_kernel(table, idx_flat)
    return out.reshape(B, L, D)


# speedup vs baseline: 1.7435x; 1.7435x over previous
import jax
import jax.numpy as jnp
from jax.experimental import pallas as pl
from jax.experimental.pallas import tpu as pltpu
from jax.experimental.pallas import tpu_sc as plsc

B = 16384
L = 50
D = 64
N = B * L

WINDOW = 128


def kernel(indices, table):
    idx_flat = indices.reshape(1, N).astype(jnp.int32)

    mesh = plsc.VectorSubcoreMesh(core_axis_name="core", subcore_axis_name="subcore")

    @pl.kernel(
        out_type=jax.ShapeDtypeStruct((N, D), table.dtype),
        mesh=mesh,
        compiler_params=pltpu.CompilerParams(use_tc_tiling_on_sc=False),
    )
    def gather_kernel(table_hbm, idx_hbm, out_hbm):
        def body(i_vmem, o_vmem):
            pltpu.sync_copy(table_hbm.at[i_vmem.at[0]], o_vmem)

        pltpu.emit_pipeline(
            body,
            grid=(N // WINDOW,),
            in_specs=[pl.BlockSpec((1, WINDOW), index_map=lambda i: (0, i))],
            out_specs=[pl.BlockSpec((WINDOW, D), index_map=lambda i: (i, 0))],
            core_axis_name=("core", "subcore"),
            dimension_semantics=(pltpu.PARALLEL,),
        )(idx_hbm, out_hbm)

    out = gather_kernel(table, idx_flat)
    return out.reshape(B, L, D)


# speedup vs baseline: 1.8706x; 1.0729x over previous
import jax
import jax.numpy as jnp
from jax.experimental import pallas as pl
from jax.experimental.pallas import tpu as pltpu
from jax.experimental.pallas import tpu_sc as plsc

B = 16384
L = 50
D = 64
N = B * L

WINDOW = 512


def kernel(indices, table):
    idx_flat = indices.reshape(1, N).astype(jnp.int32)

    mesh = plsc.VectorSubcoreMesh(core_axis_name="core", subcore_axis_name="subcore")

    @pl.kernel(
        out_type=jax.ShapeDtypeStruct((N, D), table.dtype),
        mesh=mesh,
        compiler_params=pltpu.CompilerParams(use_tc_tiling_on_sc=False),
    )
    def gather_kernel(table_hbm, idx_hbm, out_hbm):
        def body(i_vmem, o_vmem):
            pltpu.sync_copy(table_hbm.at[i_vmem.at[0]], o_vmem)

        pltpu.emit_pipeline(
            body,
            grid=(N // WINDOW,),
            in_specs=[pl.BlockSpec((1, WINDOW), index_map=lambda i: (0, i))],
            out_specs=[pl.BlockSpec((WINDOW, D), index_map=lambda i: (i, 0))],
            core_axis_name=("core", "subcore"),
            dimension_semantics=(pltpu.PARALLEL,),
        )(idx_hbm, out_hbm)

    out = gather_kernel(table, idx_flat)
    return out.reshape(B, L, D)
